# baseline (device time: 102011 ns/iter reference)
import jax
import jax.numpy as jnp
from jax import lax
from jax.experimental import pallas as pl
from jax.experimental.pallas import tpu as pltpu

N_DEV = 4
SQ = 2048
SKV = 2048
HQ_LOCAL = 8
DH = 128
D_MODEL = 1024
D_LOCAL = HQ_LOCAL * DH
SCALE = 0.08838834764831843
BLK = 64
CH = SQ // N_DEV
QC = D_MODEL // N_DEV


def kernel(x, Wq, K_ext, V_ext, Wo):
    my = lax.axis_index("i")
    xs = x[0]
    Wq_l = lax.dynamic_slice_in_dim(Wq, my * D_LOCAL, D_LOCAL, axis=1)
    Wq_l = Wq_l.astype(jnp.bfloat16)
    Wo_l = lax.dynamic_slice_in_dim(Wo, my * D_LOCAL, D_LOCAL, axis=0)
    Wo_l = Wo_l.astype(jnp.bfloat16)
    K2 = K_ext[0].reshape(SKV, D_LOCAL)
    V2 = V_ext[0].reshape(SKV, D_LOCAL)

    def body(x_ref, wq_ref, k_ref, v_ref, wo_ref, out_ref,
             ctx_ref, q_ref, xb_ref, kb_ref, vb_ref,
             rs_send_ref, rs_recv_ref, ag_send_ref,
             ag_recv_ref, rs_send_sems, rs_recv_sems, ag_send_sems,
             ag_recv_sems):
        my_pos = lax.axis_index("i")
        peers = [lax.rem(my_pos + 1 + p, N_DEV) for p in range(N_DEV - 1)]

        def entry_barrier():
            barrier_sem = pltpu.get_barrier_semaphore()
            for pr in peers:
                pl.semaphore_signal(
                    barrier_sem, inc=1,
                    device_id=(pr,), device_id_type=pl.DeviceIdType.MESH,
                )
            pl.semaphore_wait(barrier_sem, N_DEV - 1)

        my_cols = pl.ds(my_pos * QC, QC)
        pending_sends = []

        def reduce_and_broadcast(r):
            rows = pl.ds(r * CH, CH)
            acc = out_ref[rows, my_cols]
            for q in range(N_DEV - 1):
                recv = pltpu.make_async_remote_copy(
                    src_ref=rs_recv_ref.at[r, q],
                    dst_ref=rs_recv_ref.at[r, q],
                    send_sem=rs_send_sems.at[r * 3 + q],
                    recv_sem=rs_recv_sems.at[r * 3 + q],
                    device_id=(my_pos,),
                    device_id_type=pl.DeviceIdType.MESH,
                )
                recv.wait_recv()
                acc = acc + rs_recv_ref[r, q].astype(jnp.float32)
            out_ref[rows, my_cols] = acc
            ag_send_ref[r, :, :] = acc.astype(jnp.bfloat16)
            for p in range(N_DEV - 1):
                tgt = peers[p]
                rdma = pltpu.make_async_remote_copy(
                    src_ref=ag_send_ref.at[r],
                    dst_ref=ag_recv_ref.at[r, 2 - p],
                    send_sem=ag_send_sems.at[r * 3 + p],
                    recv_sem=ag_recv_sems.at[r * 3 + (2 - p)],
                    device_id=(tgt,),
                    device_id_type=pl.DeviceIdType.MESH,
                )
                rdma.start()
                pending_sends.append(rdma)

        for r in range(N_DEV):
            rows = pl.ds(r * CH, CH)
            nk = (r + 1) * CH
            xb_ref[...] = x_ref[rows, :].astype(jnp.bfloat16)
            kb_ref[rows, :] = k_ref[rows, :].astype(jnp.bfloat16)
            vb_ref[rows, :] = v_ref[rows, :].astype(jnp.bfloat16)
            q_all = jnp.dot(xb_ref[...], wq_ref[...],
                            preferred_element_type=jnp.float32)
            q_ref[...] = (q_all * SCALE).astype(jnp.bfloat16)
            qb = lax.broadcasted_iota(jnp.int32, (CH, 1), 0) // BLK \
                + r * (CH // BLK)
            kb = lax.broadcasted_iota(jnp.int32, (1, nk), 1) // BLK
            keep = kb <= qb

            def head_body(h, carry, r=r, nk=nk, keep=keep):
                c = pl.ds(h * DH, DH)
                s = lax.dot_general(q_ref[:, c], kb_ref[pl.ds(0, nk), c],
                                    (((1,), (1,)), ((), ())),
                                    preferred_element_type=jnp.float32)
                w = jnp.exp(jnp.where(keep, s, -30.0))
                denom = jnp.sum(w, axis=1, keepdims=True)
                ctx = jnp.dot(w.astype(jnp.bfloat16), vb_ref[pl.ds(0, nk), c],
                              preferred_element_type=jnp.float32)
                ctx_ref[pl.ds(r * CH, CH), c] = (ctx / denom).astype(
                    jnp.bfloat16)
                return carry

            lax.fori_loop(0, HQ_LOCAL, head_body, 0, unroll=2)
            partial_r = jnp.dot(ctx_ref[rows, :], wo_ref[...],
                                preferred_element_type=jnp.float32)
            out_ref[rows, :] = partial_r
            rs_send_ref[r, :, :] = partial_r.astype(jnp.bfloat16)
            if r == 0:
                entry_barrier()

            for p in range(N_DEV - 1):
                tgt = peers[p]
                rdma = pltpu.make_async_remote_copy(
                    src_ref=rs_send_ref.at[r, :, pl.ds(tgt * QC, QC)],
                    dst_ref=rs_recv_ref.at[r, 2 - p],
                    send_sem=rs_send_sems.at[r * 3 + p],
                    recv_sem=rs_recv_sems.at[r * 3 + (2 - p)],
                    device_id=(tgt,),
                    device_id_type=pl.DeviceIdType.MESH,
                )
                rdma.start()
                pending_sends.append(rdma)

            if r >= 1:
                reduce_and_broadcast(r - 1)

        reduce_and_broadcast(N_DEV - 1)

        for r in range(N_DEV):
            rows = pl.ds(r * CH, CH)
            for q in range(N_DEV - 1):
                recv = pltpu.make_async_remote_copy(
                    src_ref=ag_recv_ref.at[r, q],
                    dst_ref=ag_recv_ref.at[r, q],
                    send_sem=ag_send_sems.at[r * 3 + q],
                    recv_sem=ag_recv_sems.at[r * 3 + q],
                    device_id=(my_pos,),
                    device_id_type=pl.DeviceIdType.MESH,
                )
                recv.wait_recv()
                out_ref[rows, pl.ds(peers[q] * QC, QC)] = (
                    ag_recv_ref[r, q].astype(jnp.float32))

        for rdma in pending_sends:
            rdma.wait_send()

    out = pl.pallas_call(
        body,
        out_shape=jax.ShapeDtypeStruct((SQ, D_MODEL), jnp.float32),
        in_specs=[pl.BlockSpec(memory_space=pltpu.VMEM)] * 5,
        out_specs=pl.BlockSpec(memory_space=pltpu.VMEM),
        scratch_shapes=[
            pltpu.VMEM((SQ, D_LOCAL), jnp.bfloat16),
            pltpu.VMEM((CH, D_LOCAL), jnp.bfloat16),
            pltpu.VMEM((CH, D_MODEL), jnp.bfloat16),
            pltpu.VMEM((SKV, D_LOCAL), jnp.bfloat16),
            pltpu.VMEM((SKV, D_LOCAL), jnp.bfloat16),
            pltpu.VMEM((N_DEV, CH, D_MODEL), jnp.bfloat16),
            pltpu.VMEM((N_DEV, N_DEV - 1, CH, QC), jnp.bfloat16),
            pltpu.VMEM((N_DEV, CH, QC), jnp.bfloat16),
            pltpu.VMEM((N_DEV, N_DEV - 1, CH, QC), jnp.bfloat16),
            pltpu.SemaphoreType.DMA((N_DEV * 3,)),
            pltpu.SemaphoreType.DMA((N_DEV * 3,)),
            pltpu.SemaphoreType.DMA((N_DEV * 3,)),
            pltpu.SemaphoreType.DMA((N_DEV * 3,)),
        ],
        compiler_params=pltpu.CompilerParams(
            collective_id=0,
            vmem_limit_bytes=100 * 1024 * 1024,
        ),
    )(xs, Wq_l, K2, V2, Wo_l)
    return out[None]


# device time: 99459 ns/iter; 1.0257x vs baseline; 1.0257x over previous
import jax
import jax.numpy as jnp
from jax import lax
from jax.experimental import pallas as pl
from jax.experimental.pallas import tpu as pltpu

N_DEV = 4
SQ = 2048
SKV = 2048
HQ_LOCAL = 8
DH = 128
D_MODEL = 1024
D_LOCAL = HQ_LOCAL * DH
SCALE = 0.08838834764831843
BLK = 64
CH = SQ // N_DEV
QC = D_MODEL // N_DEV


def kernel(x, Wq, K_ext, V_ext, Wo):
    my = lax.axis_index("i")
    xs = x[0]
    Wq_l = lax.dynamic_slice_in_dim(Wq, my * D_LOCAL, D_LOCAL, axis=1)
    Wq_l = Wq_l.astype(jnp.bfloat16)
    Wo_l = lax.dynamic_slice_in_dim(Wo, my * D_LOCAL, D_LOCAL, axis=0)
    Wo_l = Wo_l.astype(jnp.bfloat16)
    K2 = K_ext[0].reshape(SKV, D_LOCAL)
    V2 = V_ext[0].reshape(SKV, D_LOCAL)

    def body(x_ref, wq_ref, k_ref, v_ref, wo_ref, out_ref,
             ctx_ref, q_ref, xb_ref, kb_ref, vb_ref,
             rs_send_ref, rs_recv_ref, ag_send_ref,
             ag_recv_ref, rs_send_sems, rs_recv_sems, ag_send_sems,
             ag_recv_sems):
        my_pos = lax.axis_index("i")
        peers = [lax.rem(my_pos + 1 + p, N_DEV) for p in range(N_DEV - 1)]

        def entry_barrier():
            barrier_sem = pltpu.get_barrier_semaphore()
            for pr in peers:
                pl.semaphore_signal(
                    barrier_sem, inc=1,
                    device_id=(pr,), device_id_type=pl.DeviceIdType.MESH,
                )
            pl.semaphore_wait(barrier_sem, N_DEV - 1)

        my_cols = pl.ds(my_pos * QC, QC)
        pending_sends = []

        def reduce_and_broadcast(r):
            rows = pl.ds(r * CH, CH)
            acc = out_ref[rows, my_cols]
            for q in range(N_DEV - 1):
                recv = pltpu.make_async_remote_copy(
                    src_ref=rs_recv_ref.at[r, q],
                    dst_ref=rs_recv_ref.at[r, q],
                    send_sem=rs_send_sems.at[r * 3 + q],
                    recv_sem=rs_recv_sems.at[r * 3 + q],
                    device_id=(my_pos,),
                    device_id_type=pl.DeviceIdType.MESH,
                )
                recv.wait_recv()
                acc = acc + rs_recv_ref[r, q].astype(jnp.float32)
            out_ref[rows, my_cols] = acc
            ag_send_ref[r, :, :] = acc.astype(jnp.bfloat16)
            for p in range(N_DEV - 1):
                tgt = peers[p]
                rdma = pltpu.make_async_remote_copy(
                    src_ref=ag_send_ref.at[r],
                    dst_ref=ag_recv_ref.at[r, 2 - p],
                    send_sem=ag_send_sems.at[r * 3 + p],
                    recv_sem=ag_recv_sems.at[r * 3 + (2 - p)],
                    device_id=(tgt,),
                    device_id_type=pl.DeviceIdType.MESH,
                )
                rdma.start()
                pending_sends.append(rdma)

        for r in range(N_DEV):
            rows = pl.ds(r * CH, CH)
            nk = (r + 1) * CH
            xb_ref[...] = x_ref[rows, :].astype(jnp.bfloat16)
            kb_ref[rows, :] = k_ref[rows, :].astype(jnp.bfloat16)
            vb_ref[rows, :] = v_ref[rows, :].astype(jnp.bfloat16)
            q_all = jnp.dot(xb_ref[...], wq_ref[...],
                            preferred_element_type=jnp.float32)
            q_ref[...] = (q_all * SCALE).astype(jnp.bfloat16)
            qb = lax.broadcasted_iota(jnp.int32, (CH, 1), 0) // BLK \
                + r * (CH // BLK)
            kb = lax.broadcasted_iota(jnp.int32, (1, nk), 1) // BLK
            keep = kb <= qb

            def head_body(h, carry, r=r, nk=nk, keep=keep):
                c = pl.ds(h * DH, DH)
                s = lax.dot_general(q_ref[:, c], kb_ref[pl.ds(0, nk), c],
                                    (((1,), (1,)), ((), ())),
                                    preferred_element_type=jnp.float32)
                w = jnp.exp(jnp.where(keep, s, -30.0))
                denom = jnp.sum(w, axis=1, keepdims=True)
                ctx = jnp.dot(w.astype(jnp.bfloat16), vb_ref[pl.ds(0, nk), c],
                              preferred_element_type=jnp.float32)
                ctx_ref[pl.ds(r * CH, CH), c] = (ctx / denom).astype(
                    jnp.bfloat16)
                return carry

            lax.fori_loop(0, HQ_LOCAL, head_body, 0, unroll=4)
            partial_r = jnp.dot(ctx_ref[rows, :], wo_ref[...],
                                preferred_element_type=jnp.float32)
            out_ref[rows, :] = partial_r
            rs_send_ref[r, :, :] = partial_r.astype(jnp.bfloat16)
            if r == 0:
                entry_barrier()

            for p in range(N_DEV - 1):
                tgt = peers[p]
                rdma = pltpu.make_async_remote_copy(
                    src_ref=rs_send_ref.at[r, :, pl.ds(tgt * QC, QC)],
                    dst_ref=rs_recv_ref.at[r, 2 - p],
                    send_sem=rs_send_sems.at[r * 3 + p],
                    recv_sem=rs_recv_sems.at[r * 3 + (2 - p)],
                    device_id=(tgt,),
                    device_id_type=pl.DeviceIdType.MESH,
                )
                rdma.start()
                pending_sends.append(rdma)

            if r >= 1:
                reduce_and_broadcast(r - 1)

        reduce_and_broadcast(N_DEV - 1)

        for r in range(N_DEV):
            rows = pl.ds(r * CH, CH)
            for q in range(N_DEV - 1):
                recv = pltpu.make_async_remote_copy(
                    src_ref=ag_recv_ref.at[r, q],
                    dst_ref=ag_recv_ref.at[r, q],
                    send_sem=ag_send_sems.at[r * 3 + q],
                    recv_sem=ag_recv_sems.at[r * 3 + q],
                    device_id=(my_pos,),
                    device_id_type=pl.DeviceIdType.MESH,
                )
                recv.wait_recv()
                out_ref[rows, pl.ds(peers[q] * QC, QC)] = (
                    ag_recv_ref[r, q].astype(jnp.float32))

        for rdma in pending_sends:
            rdma.wait_send()

    out = pl.pallas_call(
        body,
        out_shape=jax.ShapeDtypeStruct((SQ, D_MODEL), jnp.float32),
        in_specs=[pl.BlockSpec(memory_space=pltpu.VMEM)] * 5,
        out_specs=pl.BlockSpec(memory_space=pltpu.VMEM),
        scratch_shapes=[
            pltpu.VMEM((SQ, D_LOCAL), jnp.bfloat16),
            pltpu.VMEM((CH, D_LOCAL), jnp.bfloat16),
            pltpu.VMEM((CH, D_MODEL), jnp.bfloat16),
            pltpu.VMEM((SKV, D_LOCAL), jnp.bfloat16),
            pltpu.VMEM((SKV, D_LOCAL), jnp.bfloat16),
            pltpu.VMEM((N_DEV, CH, D_MODEL), jnp.bfloat16),
            pltpu.VMEM((N_DEV, N_DEV - 1, CH, QC), jnp.bfloat16),
            pltpu.VMEM((N_DEV, CH, QC), jnp.bfloat16),
            pltpu.VMEM((N_DEV, N_DEV - 1, CH, QC), jnp.bfloat16),
            pltpu.SemaphoreType.DMA((N_DEV * 3,)),
            pltpu.SemaphoreType.DMA((N_DEV * 3,)),
            pltpu.SemaphoreType.DMA((N_DEV * 3,)),
            pltpu.SemaphoreType.DMA((N_DEV * 3,)),
        ],
        compiler_params=pltpu.CompilerParams(
            collective_id=0,
            vmem_limit_bytes=100 * 1024 * 1024,
        ),
    )(xs, Wq_l, K2, V2, Wo_l)
    return out[None]


# device time: 98309 ns/iter; 1.0377x vs baseline; 1.0117x over previous
import jax
import jax.numpy as jnp
from jax import lax
from jax.experimental import pallas as pl
from jax.experimental.pallas import tpu as pltpu

N_DEV = 4
SQ = 2048
SKV = 2048
HQ_LOCAL = 8
DH = 128
D_MODEL = 1024
D_LOCAL = HQ_LOCAL * DH
SCALE = 0.08838834764831843
BLK = 64
CH = SQ // N_DEV
QC = D_MODEL // N_DEV


def kernel(x, Wq, K_ext, V_ext, Wo):
    my = lax.axis_index("i")
    xs = x[0].astype(jnp.bfloat16)
    Wq_l = lax.dynamic_slice_in_dim(Wq, my * D_LOCAL, D_LOCAL, axis=1)
    Wq_l = Wq_l.astype(jnp.bfloat16)
    Wo_l = lax.dynamic_slice_in_dim(Wo, my * D_LOCAL, D_LOCAL, axis=0)
    Wo_l = Wo_l.astype(jnp.bfloat16)
    K2 = K_ext[0].reshape(SKV, D_LOCAL)
    V2 = V_ext[0].reshape(SKV, D_LOCAL)

    def body(x_ref, wq_ref, k_ref, v_ref, wo_ref, out_ref,
             ctx_ref, q_ref, kb_ref, vb_ref, k_stage_ref, v_stage_ref,
             rs_send_ref, rs_recv_ref, ag_send_ref,
             ag_recv_ref, k_sems, v_sems,
             rs_send_sems, rs_recv_sems, ag_send_sems,
             ag_recv_sems):
        my_pos = lax.axis_index("i")
        peers = [lax.rem(my_pos + 1 + p, N_DEV) for p in range(N_DEV - 1)]

        barrier_sem = pltpu.get_barrier_semaphore()
        for pr in peers:
            pl.semaphore_signal(
                barrier_sem, inc=1,
                device_id=(pr,), device_id_type=pl.DeviceIdType.MESH,
            )
        pl.semaphore_wait(barrier_sem, N_DEV - 1)

        my_cols = pl.ds(my_pos * QC, QC)
        pending_sends = []

        def kv_dma(rr):
            rows = pl.ds(rr * CH, CH)
            slot = rr % 2
            return (
                pltpu.make_async_copy(
                    k_ref.at[rows, :], k_stage_ref.at[slot], k_sems.at[slot]),
                pltpu.make_async_copy(
                    v_ref.at[rows, :], v_stage_ref.at[slot], v_sems.at[slot]),
            )

        def start_kv(rr):
            for c in kv_dma(rr):
                c.start()

        def land_kv(rr):
            rows = pl.ds(rr * CH, CH)
            slot = rr % 2
            for c in kv_dma(rr):
                c.wait()
            kb_ref[rows, :] = k_stage_ref[slot].astype(jnp.bfloat16)
            vb_ref[rows, :] = v_stage_ref[slot].astype(jnp.bfloat16)

        def reduce_and_broadcast(r):
            rows = pl.ds(r * CH, CH)
            acc = out_ref[rows, my_cols]
            for q in range(N_DEV - 1):
                recv = pltpu.make_async_remote_copy(
                    src_ref=rs_recv_ref.at[r, q],
                    dst_ref=rs_recv_ref.at[r, q],
                    send_sem=rs_send_sems.at[r * 3 + q],
                    recv_sem=rs_recv_sems.at[r * 3 + q],
                    device_id=(my_pos,),
                    device_id_type=pl.DeviceIdType.MESH,
                )
                recv.wait_recv()
                acc = acc + rs_recv_ref[r, q].astype(jnp.float32)
            out_ref[rows, my_cols] = acc
            ag_send_ref[r, :, :] = acc.astype(jnp.bfloat16)
            for p in range(N_DEV - 1):
                tgt = peers[p]
                rdma = pltpu.make_async_remote_copy(
                    src_ref=ag_send_ref.at[r],
                    dst_ref=ag_recv_ref.at[r, 2 - p],
                    send_sem=ag_send_sems.at[r * 3 + p],
                    recv_sem=ag_recv_sems.at[r * 3 + (2 - p)],
                    device_id=(tgt,),
                    device_id_type=pl.DeviceIdType.MESH,
                )
                rdma.start()
                pending_sends.append(rdma)

        start_kv(0)
        for r in range(N_DEV):
            rows = pl.ds(r * CH, CH)
            nk = (r + 1) * CH
            if r + 1 < N_DEV:
                start_kv(r + 1)
            q_all = jnp.dot(x_ref[rows, :], wq_ref[...],
                            preferred_element_type=jnp.float32)
            q_ref[...] = (q_all * SCALE).astype(jnp.bfloat16)
            qb = lax.broadcasted_iota(jnp.int32, (CH, 1), 0) // BLK \
                + r * (CH // BLK)
            kb = lax.broadcasted_iota(jnp.int32, (1, nk), 1) // BLK
            keep = kb <= qb
            land_kv(r)

            def head_body(h, carry, r=r, nk=nk, keep=keep):
                c = pl.ds(h * DH, DH)
                s = lax.dot_general(q_ref[:, c], kb_ref[pl.ds(0, nk), c],
                                    (((1,), (1,)), ((), ())),
                                    preferred_element_type=jnp.float32)
                w = jnp.exp(jnp.where(keep, s, -30.0))
                denom = jnp.sum(w, axis=1, keepdims=True)
                ctx = jnp.dot(w.astype(jnp.bfloat16), vb_ref[pl.ds(0, nk), c],
                              preferred_element_type=jnp.float32)
                ctx_ref[pl.ds(r * CH, CH), c] = (ctx / denom).astype(
                    jnp.bfloat16)
                return carry

            lax.fori_loop(0, HQ_LOCAL, head_body, 0, unroll=4)
            partial_r = jnp.dot(ctx_ref[rows, :], wo_ref[...],
                                preferred_element_type=jnp.float32)
            out_ref[rows, :] = partial_r
            rs_send_ref[r, :, :] = partial_r.astype(jnp.bfloat16)

            for p in range(N_DEV - 1):
                tgt = peers[p]
                rdma = pltpu.make_async_remote_copy(
                    src_ref=rs_send_ref.at[r, :, pl.ds(tgt * QC, QC)],
                    dst_ref=rs_recv_ref.at[r, 2 - p],
                    send_sem=rs_send_sems.at[r * 3 + p],
                    recv_sem=rs_recv_sems.at[r * 3 + (2 - p)],
                    device_id=(tgt,),
                    device_id_type=pl.DeviceIdType.MESH,
                )
                rdma.start()
                pending_sends.append(rdma)

            if r >= 1:
                reduce_and_broadcast(r - 1)

        reduce_and_broadcast(N_DEV - 1)

        for r in range(N_DEV):
            rows = pl.ds(r * CH, CH)
            for q in range(N_DEV - 1):
                recv = pltpu.make_async_remote_copy(
                    src_ref=ag_recv_ref.at[r, q],
                    dst_ref=ag_recv_ref.at[r, q],
                    send_sem=ag_send_sems.at[r * 3 + q],
                    recv_sem=ag_recv_sems.at[r * 3 + q],
                    device_id=(my_pos,),
                    device_id_type=pl.DeviceIdType.MESH,
                )
                recv.wait_recv()
                out_ref[rows, pl.ds(peers[q] * QC, QC)] = (
                    ag_recv_ref[r, q].astype(jnp.float32))

        for rdma in pending_sends:
            rdma.wait_send()

    out = pl.pallas_call(
        body,
        out_shape=jax.ShapeDtypeStruct((SQ, D_MODEL), jnp.float32),
        in_specs=[
            pl.BlockSpec(memory_space=pltpu.VMEM),
            pl.BlockSpec(memory_space=pltpu.VMEM),
            pl.BlockSpec(memory_space=pltpu.MemorySpace.HBM),
            pl.BlockSpec(memory_space=pltpu.MemorySpace.HBM),
            pl.BlockSpec(memory_space=pltpu.VMEM),
        ],
        out_specs=pl.BlockSpec(memory_space=pltpu.VMEM),
        scratch_shapes=[
            pltpu.VMEM((SQ, D_LOCAL), jnp.bfloat16),
            pltpu.VMEM((CH, D_LOCAL), jnp.bfloat16),
            pltpu.VMEM((SKV, D_LOCAL), jnp.bfloat16),
            pltpu.VMEM((SKV, D_LOCAL), jnp.bfloat16),
            pltpu.VMEM((2, CH, D_LOCAL), jnp.float32),
            pltpu.VMEM((2, CH, D_LOCAL), jnp.float32),
            pltpu.VMEM((N_DEV, CH, D_MODEL), jnp.bfloat16),
            pltpu.VMEM((N_DEV, N_DEV - 1, CH, QC), jnp.bfloat16),
            pltpu.VMEM((N_DEV, CH, QC), jnp.bfloat16),
            pltpu.VMEM((N_DEV, N_DEV - 1, CH, QC), jnp.bfloat16),
            pltpu.SemaphoreType.DMA((2,)),
            pltpu.SemaphoreType.DMA((2,)),
            pltpu.SemaphoreType.DMA((N_DEV * 3,)),
            pltpu.SemaphoreType.DMA((N_DEV * 3,)),
            pltpu.SemaphoreType.DMA((N_DEV * 3,)),
            pltpu.SemaphoreType.DMA((N_DEV * 3,)),
        ],
        compiler_params=pltpu.CompilerParams(
            collective_id=0,
            vmem_limit_bytes=112 * 1024 * 1024,
        ),
    )(xs, Wq_l, K2, V2, Wo_l)
    return out[None]


# device time: 98243 ns/iter; 1.0384x vs baseline; 1.0007x over previous
import jax
import jax.numpy as jnp
from jax import lax
from jax.experimental import pallas as pl
from jax.experimental.pallas import tpu as pltpu

N_DEV = 4
SQ = 2048
SKV = 2048
HQ_LOCAL = 8
DH = 128
D_MODEL = 1024
D_LOCAL = HQ_LOCAL * DH
SCALE = 0.08838834764831843
BLK = 64
CH = 512
QC = D_MODEL // N_DEV

BLOCKS = [(0, 256), (256, 256), (512, 512), (1024, 512), (1536, 512)]
NB = len(BLOCKS)


def kernel(x, Wq, K_ext, V_ext, Wo):
    my = lax.axis_index("i")
    xs = x[0].astype(jnp.bfloat16)
    Wq_l = lax.dynamic_slice_in_dim(Wq, my * D_LOCAL, D_LOCAL, axis=1)
    Wq_l = Wq_l.astype(jnp.bfloat16)
    Wo_l = lax.dynamic_slice_in_dim(Wo, my * D_LOCAL, D_LOCAL, axis=0)
    Wo_l = Wo_l.astype(jnp.bfloat16)
    K2 = K_ext[0].reshape(SKV, D_LOCAL).astype(jnp.bfloat16)
    V2 = V_ext[0].reshape(SKV, D_LOCAL).astype(jnp.bfloat16)

    def body(x_ref, wq_ref, k_ref, v_ref, wo_ref, out_ref,
             ctx_ref, q_ref, rs_send_ref, rs_recv_ref, ag_send_ref,
             ag_recv_ref, rs_send_sems, rs_recv_sems, ag_send_sems,
             ag_recv_sems):
        my_pos = lax.axis_index("i")
        peers = [lax.rem(my_pos + 1 + p, N_DEV) for p in range(N_DEV - 1)]

        barrier_sem = pltpu.get_barrier_semaphore()
        for pr in peers:
            pl.semaphore_signal(
                barrier_sem, inc=1,
                device_id=(pr,), device_id_type=pl.DeviceIdType.MESH,
            )
        pl.semaphore_wait(barrier_sem, N_DEV - 1)

        my_cols = pl.ds(my_pos * QC, QC)
        pending_sends = []

        def reduce_and_broadcast(r):
            start, size = BLOCKS[r]
            rows = pl.ds(start, size)
            brows = pl.ds(0, size)
            acc = out_ref[rows, my_cols]
            for q in range(N_DEV - 1):
                recv = pltpu.make_async_remote_copy(
                    src_ref=rs_recv_ref.at[r, q, brows, :],
                    dst_ref=rs_recv_ref.at[r, q, brows, :],
                    send_sem=rs_send_sems.at[r * 3 + q],
                    recv_sem=rs_recv_sems.at[r * 3 + q],
                    device_id=(my_pos,),
                    device_id_type=pl.DeviceIdType.MESH,
                )
                recv.wait_recv()
                acc = acc + rs_recv_ref[r, q, brows, :].astype(jnp.float32)
            out_ref[rows, my_cols] = acc
            ag_send_ref[r, brows, :] = acc.astype(jnp.bfloat16)
            for p in range(N_DEV - 1):
                tgt = peers[p]
                rdma = pltpu.make_async_remote_copy(
                    src_ref=ag_send_ref.at[r, brows, :],
                    dst_ref=ag_recv_ref.at[r, 2 - p, brows, :],
                    send_sem=ag_send_sems.at[r * 3 + p],
                    recv_sem=ag_recv_sems.at[r * 3 + (2 - p)],
                    device_id=(tgt,),
                    device_id_type=pl.DeviceIdType.MESH,
                )
                rdma.start()
                pending_sends.append(rdma)

        for r, (start, size) in enumerate(BLOCKS):
            rows = pl.ds(start, size)
            brows = pl.ds(0, size)
            nk = start + size
            q_all = jnp.dot(x_ref[rows, :], wq_ref[...],
                            preferred_element_type=jnp.float32)
            q_ref[brows, :] = (q_all * SCALE).astype(jnp.bfloat16)
            qb = lax.broadcasted_iota(jnp.int32, (size, 1), 0) // BLK \
                + start // BLK
            kb = lax.broadcasted_iota(jnp.int32, (1, nk), 1) // BLK
            keep = kb <= qb

            def head_body(h, carry, start=start, size=size, nk=nk,
                          keep=keep, brows=brows):
                c = pl.ds(h * DH, DH)
                s = lax.dot_general(q_ref[brows, c], k_ref[pl.ds(0, nk), c],
                                    (((1,), (1,)), ((), ())),
                                    preferred_element_type=jnp.float32)
                w = jnp.exp(jnp.where(keep, s, -30.0))
                denom = jnp.sum(w, axis=1, keepdims=True)
                ctx = jnp.dot(w.astype(jnp.bfloat16), v_ref[pl.ds(0, nk), c],
                              preferred_element_type=jnp.float32)
                ctx_ref[pl.ds(start, size), c] = (ctx / denom).astype(
                    jnp.bfloat16)
                return carry

            lax.fori_loop(0, HQ_LOCAL, head_body, 0, unroll=4)
            partial_r = jnp.dot(ctx_ref[rows, :], wo_ref[...],
                                preferred_element_type=jnp.float32)
            out_ref[rows, :] = partial_r
            rs_send_ref[r, brows, :] = partial_r.astype(jnp.bfloat16)

            for p in range(N_DEV - 1):
                tgt = peers[p]
                rdma = pltpu.make_async_remote_copy(
                    src_ref=rs_send_ref.at[r, brows, pl.ds(tgt * QC, QC)],
                    dst_ref=rs_recv_ref.at[r, 2 - p, brows, :],
                    send_sem=rs_send_sems.at[r * 3 + p],
                    recv_sem=rs_recv_sems.at[r * 3 + (2 - p)],
                    device_id=(tgt,),
                    device_id_type=pl.DeviceIdType.MESH,
                )
                rdma.start()
                pending_sends.append(rdma)

            if r >= 1:
                reduce_and_broadcast(r - 1)

        reduce_and_broadcast(NB - 1)

        for r, (start, size) in enumerate(BLOCKS):
            rows = pl.ds(start, size)
            brows = pl.ds(0, size)
            for q in range(N_DEV - 1):
                recv = pltpu.make_async_remote_copy(
                    src_ref=ag_recv_ref.at[r, q, brows, :],
                    dst_ref=ag_recv_ref.at[r, q, brows, :],
                    send_sem=ag_send_sems.at[r * 3 + q],
                    recv_sem=ag_recv_sems.at[r * 3 + q],
                    device_id=(my_pos,),
                    device_id_type=pl.DeviceIdType.MESH,
                )
                recv.wait_recv()
                out_ref[rows, pl.ds(peers[q] * QC, QC)] = (
                    ag_recv_ref[r, q, brows, :].astype(jnp.float32))

        for rdma in pending_sends:
            rdma.wait_send()

    out = pl.pallas_call(
        body,
        out_shape=jax.ShapeDtypeStruct((SQ, D_MODEL), jnp.float32),
        in_specs=[pl.BlockSpec(memory_space=pltpu.VMEM)] * 5,
        out_specs=pl.BlockSpec(memory_space=pltpu.VMEM),
        scratch_shapes=[
            pltpu.VMEM((SQ, D_LOCAL), jnp.bfloat16),
            pltpu.VMEM((CH, D_LOCAL), jnp.bfloat16),
            pltpu.VMEM((NB, CH, D_MODEL), jnp.bfloat16),
            pltpu.VMEM((NB, N_DEV - 1, CH, QC), jnp.bfloat16),
            pltpu.VMEM((NB, CH, QC), jnp.bfloat16),
            pltpu.VMEM((NB, N_DEV - 1, CH, QC), jnp.bfloat16),
            pltpu.SemaphoreType.DMA((NB * 3,)),
            pltpu.SemaphoreType.DMA((NB * 3,)),
            pltpu.SemaphoreType.DMA((NB * 3,)),
            pltpu.SemaphoreType.DMA((NB * 3,)),
        ],
        compiler_params=pltpu.CompilerParams(
            collective_id=0,
            vmem_limit_bytes=100 * 1024 * 1024,
        ),
    )(xs, Wq_l, K2, V2, Wo_l)
    return out[None]


# device time: 95801 ns/iter; 1.0648x vs baseline; 1.0255x over previous
import jax
import jax.numpy as jnp
from jax import lax
from jax.experimental import pallas as pl
from jax.experimental.pallas import tpu as pltpu

N_DEV = 4
SQ = 2048
SKV = 2048
HQ_LOCAL = 8
DH = 128
D_MODEL = 1024
D_LOCAL = HQ_LOCAL * DH
SCALE = 0.08838834764831843
BLK = 64
CH = SQ // N_DEV
QC = D_MODEL // N_DEV


def kernel(x, Wq, K_ext, V_ext, Wo):
    my = lax.axis_index("i")
    xs = x[0].astype(jnp.bfloat16)
    Wq_l = lax.dynamic_slice_in_dim(Wq, my * D_LOCAL, D_LOCAL, axis=1)
    Wq_l = Wq_l.astype(jnp.bfloat16)
    Wo_l = lax.dynamic_slice_in_dim(Wo, my * D_LOCAL, D_LOCAL, axis=0)
    Wo_l = Wo_l.astype(jnp.bfloat16)
    K2 = K_ext[0].reshape(SKV, D_LOCAL).astype(jnp.bfloat16)
    V2 = V_ext[0].reshape(SKV, D_LOCAL).astype(jnp.bfloat16)

    def body(x_ref, wq_ref, k_ref, v_ref, wo_ref, out_ref,
             ctx_ref, q_ref, rs_send_ref, rs_recv_ref, ag_send_ref,
             ag_recv_ref, rs_send_sems, rs_recv_sems, ag_send_sems,
             ag_recv_sems):
        my_pos = lax.axis_index("i")
        peers = [lax.rem(my_pos + 1 + p, N_DEV) for p in range(N_DEV - 1)]

        barrier_sem = pltpu.get_barrier_semaphore()
        for pr in peers:
            pl.semaphore_signal(
                barrier_sem, inc=1,
                device_id=(pr,), device_id_type=pl.DeviceIdType.MESH,
            )
        pl.semaphore_wait(barrier_sem, N_DEV - 1)

        my_cols = pl.ds(my_pos * QC, QC)
        pending_sends = []

        def reduce_and_broadcast(r):
            rows = pl.ds(r * CH, CH)
            acc = out_ref[rows, my_cols]
            for q in range(N_DEV - 1):
                recv = pltpu.make_async_remote_copy(
                    src_ref=rs_recv_ref.at[r, q],
                    dst_ref=rs_recv_ref.at[r, q],
                    send_sem=rs_send_sems.at[r * 3 + q],
                    recv_sem=rs_recv_sems.at[r * 3 + q],
                    device_id=(my_pos,),
                    device_id_type=pl.DeviceIdType.MESH,
                )
                recv.wait_recv()
                acc = acc + rs_recv_ref[r, q].astype(jnp.float32)
            out_ref[rows, my_cols] = acc
            ag_send_ref[r, :, :] = acc.astype(jnp.bfloat16)
            for p in range(N_DEV - 1):
                tgt = peers[p]
                rdma = pltpu.make_async_remote_copy(
                    src_ref=ag_send_ref.at[r],
                    dst_ref=ag_recv_ref.at[r, 2 - p],
                    send_sem=ag_send_sems.at[r * 3 + p],
                    recv_sem=ag_recv_sems.at[r * 3 + (2 - p)],
                    device_id=(tgt,),
                    device_id_type=pl.DeviceIdType.MESH,
                )
                rdma.start()
                pending_sends.append(rdma)

        for r in range(N_DEV):
            rows = pl.ds(r * CH, CH)
            nk = (r + 1) * CH
            q_all = jnp.dot(x_ref[rows, :], wq_ref[...],
                            preferred_element_type=jnp.float32)
            q_ref[...] = (q_all * SCALE).astype(jnp.bfloat16)
            qb = lax.broadcasted_iota(jnp.int32, (CH, 1), 0) // BLK \
                + r * (CH // BLK)
            kb = lax.broadcasted_iota(jnp.int32, (1, nk), 1) // BLK
            keep = kb <= qb

            def head_body(h, carry, r=r, nk=nk, keep=keep):
                c = pl.ds(h * DH, DH)
                s = lax.dot_general(q_ref[:, c], k_ref[pl.ds(0, nk), c],
                                    (((1,), (1,)), ((), ())),
                                    preferred_element_type=jnp.float32)
                w = jnp.exp(jnp.where(keep, s, -30.0))
                denom = jnp.sum(w, axis=1, keepdims=True)
                ctx = jnp.dot(w.astype(jnp.bfloat16), v_ref[pl.ds(0, nk), c],
                              preferred_element_type=jnp.float32)
                ctx_ref[pl.ds(r * CH, CH), c] = (ctx / denom).astype(
                    jnp.bfloat16)
                return carry

            lax.fori_loop(0, HQ_LOCAL, head_body, 0, unroll=4)
            partial_r = jnp.dot(ctx_ref[rows, :], wo_ref[...],
                                preferred_element_type=jnp.float32)
            out_ref[rows, :] = partial_r
            rs_send_ref[r, :, :] = partial_r.astype(jnp.bfloat16)

            for p in range(N_DEV - 1):
                tgt = peers[p]
                rdma = pltpu.make_async_remote_copy(
                    src_ref=rs_send_ref.at[r, :, pl.ds(tgt * QC, QC)],
                    dst_ref=rs_recv_ref.at[r, 2 - p],
                    send_sem=rs_send_sems.at[r * 3 + p],
                    recv_sem=rs_recv_sems.at[r * 3 + (2 - p)],
                    device_id=(tgt,),
                    device_id_type=pl.DeviceIdType.MESH,
                )
                rdma.start()
                pending_sends.append(rdma)

            if r >= 1:
                reduce_and_broadcast(r - 1)

        reduce_and_broadcast(N_DEV - 1)

        for r in range(N_DEV):
            rows = pl.ds(r * CH, CH)
            for q in range(N_DEV - 1):
                recv = pltpu.make_async_remote_copy(
                    src_ref=ag_recv_ref.at[r, q],
                    dst_ref=ag_recv_ref.at[r, q],
                    send_sem=ag_send_sems.at[r * 3 + q],
                    recv_sem=ag_recv_sems.at[r * 3 + q],
                    device_id=(my_pos,),
                    device_id_type=pl.DeviceIdType.MESH,
                )
                recv.wait_recv()
                out_ref[rows, pl.ds(peers[q] * QC, QC)] = (
                    ag_recv_ref[r, q].astype(jnp.float32))

        for rdma in pending_sends:
            rdma.wait_send()

    out = pl.pallas_call(
        body,
        out_shape=jax.ShapeDtypeStruct((SQ, D_MODEL), jnp.float32),
        in_specs=[pl.BlockSpec(memory_space=pltpu.VMEM)] * 5,
        out_specs=pl.BlockSpec(memory_space=pltpu.VMEM),
        scratch_shapes=[
            pltpu.VMEM((SQ, D_LOCAL), jnp.bfloat16),
            pltpu.VMEM((CH, D_LOCAL), jnp.bfloat16),
            pltpu.VMEM((N_DEV, CH, D_MODEL), jnp.bfloat16),
            pltpu.VMEM((N_DEV, N_DEV - 1, CH, QC), jnp.bfloat16),
            pltpu.VMEM((N_DEV, CH, QC), jnp.bfloat16),
            pltpu.VMEM((N_DEV, N_DEV - 1, CH, QC), jnp.bfloat16),
            pltpu.SemaphoreType.DMA((N_DEV * 3,)),
            pltpu.SemaphoreType.DMA((N_DEV * 3,)),
            pltpu.SemaphoreType.DMA((N_DEV * 3,)),
            pltpu.SemaphoreType.DMA((N_DEV * 3,)),
        ],
        compiler_params=pltpu.CompilerParams(
            collective_id=0,
            vmem_limit_bytes=100 * 1024 * 1024,
        ),
    )(xs, Wq_l, K2, V2, Wo_l)
    return out[None]
